# baseline (speedup 1.0000x reference)
import functools

import jax
import jax.numpy as jnp
from jax import lax
from jax.experimental import pallas as pl
from jax.experimental.pallas import tpu as pltpu
from jax.experimental.pallas import tpu_sc as plsc

NC = 2
NS = 16
N_NODES = 10000
N_EDGES = 320000
D = 128
DH = D // NC
EPT = N_EDGES // NS
K = 125
CHUNKS = EPT // K
N_PAD = 10240
RPT = N_PAD // NS
RING = 3


def _sc_aggregate(xh, ei):
  mesh = plsc.VectorSubcoreMesh(core_axis_name="c", subcore_axis_name="s")

  @functools.partial(
      pl.kernel,
      out_type=jax.ShapeDtypeStruct((NC, N_PAD, DH), jnp.float32),
      mesh=mesh,
      compiler_params=pltpu.CompilerParams(use_tc_tiling_on_sc=False),
      scratch_types=[
          pltpu.VMEM_SHARED((N_PAD, DH), jnp.float32),
          pltpu.VMEM_SHARED((N_PAD, DH), jnp.float32),
          pltpu.VMEM((RING, 2, K), jnp.int32),
          pltpu.VMEM((RING, K, DH), jnp.float32),
          pltpu.SemaphoreType.DMA,
          pltpu.SemaphoreType.DMA,
      ],
  )
  def body(xh_hbm, ei_hbm, out_hbm, xsh, acc, idx4, rows, gsem, isem):
    c = lax.axis_index("c")
    s = lax.axis_index("s")
    pltpu.sync_copy(xh_hbm.at[c, pl.ds(s * RPT, RPT)],
                    xsh.at[pl.ds(s * RPT, RPT)])
    zero16 = jnp.zeros((16,), jnp.float32)
    for r in range(K):
      for j in range(DH // 16):
        rows[0, r, pl.ds(j * 16, 16)] = zero16
    for j in range(5):
      pltpu.sync_copy(rows.at[0, pl.ds(0, 120)],
                      acc.at[pl.ds(s * RPT + j * 120, 120)])
    pltpu.sync_copy(rows.at[0, pl.ds(0, 40)],
                    acc.at[pl.ds(s * RPT + 600, 40)])
    plsc.subcore_barrier()

    for p in range(RING - 1):
      pltpu.sync_copy(ei_hbm.at[s, p], idx4.at[p])
      pltpu.async_copy(xsh.at[idx4.at[p, 0]], rows.at[p], gsem)
    pltpu.async_copy(ei_hbm.at[s, RING - 1], idx4.at[RING - 1], isem)

    def chunk(i, carry):
      b = lax.rem(i, RING)
      pltpu.make_async_copy(xsh.at[idx4.at[b, 0]], rows.at[b], gsem).wait()

      @pl.when(i + RING - 1 < CHUNKS)
      def _():
        b2 = lax.rem(i + RING - 1, RING)
        pltpu.make_async_copy(ei_hbm.at[s, i + RING - 1], idx4.at[b2],
                              isem).wait()
        pltpu.async_copy(xsh.at[idx4.at[b2, 0]], rows.at[b2], gsem)

      pltpu.sync_copy(rows.at[b], acc.at[idx4.at[b, 1]], add=True)

      @pl.when(i + RING < CHUNKS)
      def _():
        pltpu.async_copy(ei_hbm.at[s, i + RING], idx4.at[b], isem)

      return carry

    lax.fori_loop(0, CHUNKS, chunk, 0)
    plsc.subcore_barrier()
    pltpu.sync_copy(acc.at[pl.ds(s * RPT, RPT)],
                    out_hbm.at[c, pl.ds(s * RPT, RPT)])

  return body(xh, ei)


def _linear_body(a_ref, w_ref, b_ref, o_ref):
  y0 = lax.dot_general(a_ref[0], w_ref[pl.ds(0, DH), :],
                       (((1,), (0,)), ((), ())),
                       preferred_element_type=jnp.float32,
                       precision=lax.Precision.HIGHEST)
  y1 = lax.dot_general(a_ref[1], w_ref[pl.ds(DH, DH), :],
                       (((1,), (0,)), ((), ())),
                       preferred_element_type=jnp.float32,
                       precision=lax.Precision.HIGHEST)
  o_ref[...] = jnp.maximum(y0 + y1 + b_ref[...], 0.0)


def _tc_linear(agg2, wt, b2):
  rb = 2000
  return pl.pallas_call(
      _linear_body,
      out_shape=jax.ShapeDtypeStruct((N_NODES, D), jnp.float32),
      grid=(N_NODES // rb,),
      in_specs=[
          pl.BlockSpec((NC, rb, DH), lambda i: (0, i, 0)),
          pl.BlockSpec((D, D), lambda i: (0, 0)),
          pl.BlockSpec((1, D), lambda i: (0, 0)),
      ],
      out_specs=pl.BlockSpec((rb, D), lambda i: (i, 0)),
  )(agg2, wt, b2)


@jax.jit
def kernel(x, edge_index, W, b):
  xpad = jnp.pad(x, ((0, N_PAD - N_NODES), (0, 0)))
  xh = jnp.transpose(xpad.reshape(N_PAD, NC, DH), (1, 0, 2))
  ei = edge_index.astype(jnp.int32).reshape(2, NS, CHUNKS, K)
  ei = jnp.transpose(ei, (1, 2, 0, 3))
  agg2 = _sc_aggregate(xh, ei)
  return _tc_linear(agg2, W.T, b.reshape(1, D))


# speedup vs baseline: 1.7934x; 1.7934x over previous
import functools

import jax
import jax.numpy as jnp
from jax import lax
from jax.experimental import pallas as pl
from jax.experimental.pallas import tpu as pltpu
from jax.experimental.pallas import tpu_sc as plsc

NC = 2
NS = 16
NW = NC * NS
N_NODES = 10000
N_EDGES = 320000
D = 128
EPW = N_EDGES // NW
K = 125
CHUNKS = EPW // K
N_PAD = 10112
RPT = N_PAD // NS


def _sc_aggregate(x, ei):
  mesh = plsc.VectorSubcoreMesh(core_axis_name="c", subcore_axis_name="s")

  @functools.partial(
      pl.kernel,
      out_type=jax.ShapeDtypeStruct((NC, N_PAD, D), jnp.float32),
      mesh=mesh,
      scratch_types=[
          pltpu.VMEM_SHARED((N_PAD, D), jnp.float32),
          pltpu.VMEM((3, 2, K), jnp.int32),
          pltpu.VMEM((3, K, D), jnp.float32),
          pltpu.SemaphoreType.DMA,
          pltpu.SemaphoreType.DMA,
      ],
  )
  def body(x_hbm, ei_hbm, out_hbm, acc, idx3, rows, gsem, isem):
    c = lax.axis_index("c")
    s = lax.axis_index("s")
    wid = s * NC + c
    for p in range(3):
      pltpu.sync_copy(ei_hbm.at[wid, p], idx3.at[p])
    pltpu.async_copy(x_hbm.at[idx3.at[0, 0]], rows.at[0], gsem)
    pltpu.async_copy(x_hbm.at[idx3.at[1, 0]], rows.at[1], gsem)
    zero16 = jnp.zeros((16,), jnp.float32)

    def zstore(i, carry):
      rows[2, i // 8, pl.ds(lax.rem(i, 8) * 16, 16)] = zero16
      return carry

    lax.fori_loop(0, 64 * 8, zstore, 0)
    for j in range(9):
      pltpu.sync_copy(rows.at[2, pl.ds(0, 64)],
                      acc.at[pl.ds(s * RPT + j * 64, 64)])
    pltpu.sync_copy(rows.at[2, pl.ds(0, 56)],
                    acc.at[pl.ds(s * RPT + 576, 56)])
    plsc.subcore_barrier()

    def chunk(i, carry):
      b = lax.rem(i, 3)
      pltpu.make_async_copy(x_hbm.at[idx3.at[b, 0]], rows.at[b], gsem).wait()

      @pl.when(jnp.logical_and(i > 0, i + 2 < CHUNKS))
      def _():
        b2 = lax.rem(i + 2, 3)
        pltpu.make_async_copy(ei_hbm.at[wid, i + 2], idx3.at[b2], isem).wait()

      @pl.when(i + 2 < CHUNKS)
      def _():
        b2 = lax.rem(i + 2, 3)
        pltpu.async_copy(x_hbm.at[idx3.at[b2, 0]], rows.at[b2], gsem)

      pltpu.sync_copy(rows.at[b], acc.at[idx3.at[b, 1]], add=True)

      @pl.when(i + 3 < CHUNKS)
      def _():
        pltpu.async_copy(ei_hbm.at[wid, i + 3], idx3.at[b], isem)

      return carry

    lax.fori_loop(0, CHUNKS, chunk, 0)
    plsc.subcore_barrier()
    pltpu.sync_copy(acc.at[pl.ds(s * RPT, RPT)],
                    out_hbm.at[c, pl.ds(s * RPT, RPT)])

  return body(x, ei)


def _linear_body(a_ref, w_ref, b_ref, o_ref):
  z = a_ref[0] + a_ref[1]
  y = lax.dot_general(z, w_ref[...], (((1,), (0,)), ((), ())),
                      preferred_element_type=jnp.float32,
                      precision=lax.Precision.HIGHEST)
  o_ref[...] = jnp.maximum(y + b_ref[...], 0.0)


def _tc_linear(agg2, wt, b2):
  rb = 2000
  return pl.pallas_call(
      _linear_body,
      out_shape=jax.ShapeDtypeStruct((N_NODES, D), jnp.float32),
      grid=(N_NODES // rb,),
      in_specs=[
          pl.BlockSpec((NC, rb, D), lambda i: (0, i, 0)),
          pl.BlockSpec((D, D), lambda i: (0, 0)),
          pl.BlockSpec((1, D), lambda i: (0, 0)),
      ],
      out_specs=pl.BlockSpec((rb, D), lambda i: (i, 0)),
  )(agg2, wt, b2)


@jax.jit
def kernel(x, edge_index, W, b):
  ei = edge_index.astype(jnp.int32).reshape(2, NW, CHUNKS, K)
  ei = jnp.transpose(ei, (1, 2, 0, 3))
  agg2 = _sc_aggregate(x, ei)
  return _tc_linear(agg2, W.T, b.reshape(1, D))
